# baseline (speedup 1.0000x reference)
import functools
import jax
import jax.numpy as jnp
from jax.experimental import pallas as pl

N_EDGES = 800000
BATCH = 16
NUM_RES_TYPES = 20
NUM_RELATION = 7
MAX_SEQ_DIST = 10
FDIM = 2 * NUM_RES_TYPES + NUM_RELATION + (MAX_SEQ_DIST + 1) + 1

BLK = 1600
NB = N_EDGES // BLK


def _feature_body(tin_ref, tout_ref, rel_ref, seq_ref, dx_ref, dy_ref, dz_ref,
                  estart_ref, nstart_ref, feat_ref, off_ref):
    tin = tin_ref[0, 0, :].reshape(BLK, 1)
    tout = tout_ref[0, 0, :].reshape(BLK, 1)
    rel = rel_ref[0, 0, :].reshape(BLK, 1)
    seq = seq_ref[0, 0, :].reshape(BLK, 1)
    dx = dx_ref[0, 0, :].reshape(BLK, 1)
    dy = dy_ref[0, 0, :].reshape(BLK, 1)
    dz = dz_ref[0, 0, :].reshape(BLK, 1)

    cols = jax.lax.broadcasted_iota(jnp.int32, (1, FDIM), 1)
    onehot = ((cols == tin) | (cols == tout + NUM_RES_TYPES)
              | (cols == rel + 2 * NUM_RES_TYPES)
              | (cols == seq + 2 * NUM_RES_TYPES + NUM_RELATION))
    sp = jnp.sqrt(dx * dx + dy * dy + dz * dz + 1e-12)
    feat = jnp.where(cols == FDIM - 1, sp, onehot.astype(jnp.float32))
    feat_ref[:, :] = feat

    i = pl.program_id(0)
    j = i * BLK + jax.lax.broadcasted_iota(jnp.int32, (BLK, 1), 0)
    estart = estart_ref[0, 0, :].reshape(1, BATCH)
    nstart = nstart_ref[0, 0, :].reshape(1, BATCH)
    g = jnp.sum((j >= estart).astype(jnp.int32), axis=1, keepdims=True) - 1
    gcols = jax.lax.broadcasted_iota(jnp.int32, (1, BATCH), 1)
    off = jnp.sum(jnp.where(gcols == g, nstart, 0), axis=1)
    off_ref[0, 0, :] = off


def _r3(x):
    return x.reshape(NB, 1, BLK)


@jax.jit
def kernel(node_position, atom2residue, residue_type, node2graph, edge_index, edge_rel):
    node_in0 = edge_index[0]
    node_out0 = edge_index[1]
    edge2graph = node2graph[node_in0]
    order = jnp.argsort(edge2graph)
    nin = node_in0[order]
    nout = node_out0[order]
    r = edge_rel[order]

    num_edges = jnp.bincount(edge2graph, length=BATCH).astype(jnp.int32)
    num_nodes = jnp.bincount(node2graph, length=BATCH).astype(jnp.int32)
    nstart = jnp.cumsum(num_nodes) - num_nodes
    estart = jnp.cumsum(num_edges) - num_edges

    rin = atom2residue[nin]
    rout = atom2residue[nout]
    t_in = residue_type[rin]
    t_out = residue_type[rout]
    seqd = jnp.clip(jnp.abs(rin - rout), 0, MAX_SEQ_DIST)
    pin = node_position[nin]
    pout = node_position[nout]
    d = pin - pout

    spec1 = pl.BlockSpec((1, 1, BLK), lambda i: (i, 0, 0))
    spec16 = pl.BlockSpec((1, 1, BATCH), lambda i: (0, 0, 0))
    feat, off3 = pl.pallas_call(
        _feature_body,
        grid=(NB,),
        in_specs=[spec1, spec1, spec1, spec1, spec1, spec1, spec1, spec16, spec16],
        out_specs=[pl.BlockSpec((BLK, FDIM), lambda i: (i, 0)), spec1],
        out_shape=[
            jax.ShapeDtypeStruct((N_EDGES, FDIM), jnp.float32),
            jax.ShapeDtypeStruct((NB, 1, BLK), jnp.int32),
        ],
    )(_r3(t_in), _r3(t_out), _r3(r), _r3(seqd),
      _r3(d[:, 0]), _r3(d[:, 1]), _r3(d[:, 2]),
      estart.reshape(1, 1, BATCH), nstart.reshape(1, 1, BATCH))

    edge_list = jnp.stack([nin, nout, r], axis=1)
    return edge_list, feat, off3.reshape(N_EDGES), num_edges
